# Initial kernel scaffold; baseline (speedup 1.0000x reference)
import jax
import jax.numpy as jnp
from jax.experimental import pallas as pl


def kernel(x, table):
    raise NotImplementedError("write your pallas kernel here")


# speedup vs baseline: 1.0954x; 1.0954x over previous
import functools

import jax
import jax.numpy as jnp
from jax import lax
from jax.experimental import pallas as pl
from jax.experimental.pallas import tpu as pltpu
from jax.experimental.pallas import tpu_sc as plsc

DATA_DIM = 1000000
D = 32
BATCH = 16384
HIST = 50
N = BATCH * HIST
NUM_WORKERS = 32
PER_WORKER = N // NUM_WORKERS
CHUNK = 1024
NUM_CHUNKS = PER_WORKER // CHUNK

_mesh = plsc.VectorSubcoreMesh(core_axis_name="c", subcore_axis_name="s")


@functools.partial(
    pl.kernel,
    mesh=_mesh,
    out_type=jax.ShapeDtypeStruct((N, D), jnp.float32),
    scratch_types=[
        pltpu.VMEM((CHUNK,), jnp.int32),
        pltpu.VMEM((CHUNK, D), jnp.float32),
        pltpu.SemaphoreType.DMA,
    ],
    compiler_params=pltpu.CompilerParams(use_tc_tiling_on_sc=False),
)
def _gather_all(idx_hbm, table_hbm, out_hbm, idx_v, rows_v, sem):
    wid = lax.axis_index("s") * 2 + lax.axis_index("c")
    base = wid * PER_WORKER

    def body(c, carry):
        off = pl.multiple_of(base + c * CHUNK, CHUNK)
        pltpu.sync_copy(idx_hbm.at[pl.ds(off, CHUNK)], idx_v)
        pltpu.async_copy(table_hbm.at[idx_v], rows_v, sem).wait()
        pltpu.sync_copy(rows_v, out_hbm.at[pl.ds(off, CHUNK)])
        return carry

    lax.fori_loop(0, NUM_CHUNKS, body, 0)


def kernel(x, table):
    idx = x.reshape(N).astype(jnp.int32)
    out = _gather_all(idx, table)
    return out.reshape(BATCH, HIST, D)


# speedup vs baseline: 1.4514x; 1.3250x over previous
import functools

import jax
import jax.numpy as jnp
from jax import lax
from jax.experimental import pallas as pl
from jax.experimental.pallas import tpu as pltpu
from jax.experimental.pallas import tpu_sc as plsc

DATA_DIM = 1000000
D = 32
BATCH = 16384
HIST = 50
N = BATCH * HIST
NUM_WORKERS = 32
BLK = 128
NUM_BLOCKS = N // BLK
BLOCKS_PER_W = NUM_BLOCKS // NUM_WORKERS
RROWS = DATA_DIM // 4

_mesh = plsc.VectorSubcoreMesh(core_axis_name="c", subcore_axis_name="s")


@functools.partial(
    pl.kernel,
    mesh=_mesh,
    out_type=jax.ShapeDtypeStruct((HIST, D, BATCH), jnp.float32),
    scratch_types=[
        pltpu.VMEM((2, BLK), jnp.int32),
        pltpu.VMEM((2, BLK), jnp.int32),
        pltpu.VMEM((2, BLK, 128), jnp.float32),
        pltpu.VMEM((D, BLK), jnp.float32),
        pltpu.SemaphoreType.DMA,
        pltpu.SemaphoreType.DMA,
    ],
    compiler_params=pltpu.CompilerParams(needs_layout_passes=False),
)
def _gather_all(xt_hbm, r_hbm, out_hbm, j_v, o_v, rows_v, stage_v, sem0, sem1):
    wid = lax.axis_index("s") * 2 + lax.axis_index("c")
    k0 = wid * BLOCKS_PER_W
    sems = (sem0, sem1)

    def issue(k, buf):
        h = k // 128
        c = k % 128
        pltpu.sync_copy(xt_hbm.at[h, pl.ds(c * BLK, BLK)], j_v.at[buf])
        for t in range(BLK // 16):
            iv = j_v[buf, pl.ds(t * 16, 16)]
            o_v[buf, pl.ds(t * 16, 16)] = (iv & 3) << 5
            j_v[buf, pl.ds(t * 16, 16)] = lax.shift_right_logical(iv, 2)
        pltpu.async_copy(r_hbm.at[j_v.at[buf]], rows_v.at[buf], sems[buf])

    def drain(k, buf):
        h = k // 128
        c = k % 128
        pltpu.make_async_copy(
            r_hbm.at[j_v.at[buf]], rows_v.at[buf], sems[buf]
        ).wait()

        def extract(t, _):
            rt = jax.lax.broadcasted_iota(jnp.int32, (16,), 0) + t * 16
            cb = o_v[buf, pl.ds(t * 16, 16)]
            for d in range(D):
                stage_v[d, pl.ds(t * 16, 16)] = plsc.load_gather(
                    rows_v.at[buf], [rt, cb + d]
                )
            return 0

        lax.fori_loop(0, BLK // 16, extract, 0, unroll=False)
        pltpu.sync_copy(stage_v, out_hbm.at[h, :, pl.ds(c * BLK, BLK)])

    issue(k0, 0)

    def body(g, _):
        k = k0 + g * 2
        issue(k + 1, 1)
        drain(k, 0)
        issue(k + 2, 0)
        drain(k + 1, 1)
        return 0

    lax.fori_loop(0, (BLOCKS_PER_W - 2) // 2, body, 0, unroll=False)
    issue(k0 + BLOCKS_PER_W - 1, 1)
    drain(k0 + BLOCKS_PER_W - 2, 0)
    drain(k0 + BLOCKS_PER_W - 1, 1)


def kernel(x, table):
    r = table.reshape(RROWS, 128)
    out = _gather_all(x.T, r)
    return out.transpose(2, 0, 1)
